# Initial kernel scaffold; baseline (speedup 1.0000x reference)
import jax
import jax.numpy as jnp
from jax.experimental import pallas as pl


def kernel(input, index):
    raise NotImplementedError("write your pallas kernel here")


# speedup vs baseline: 11.4585x; 11.4585x over previous
import jax
import jax.numpy as jnp
from jax import lax
from jax.experimental import pallas as pl
from jax.experimental.pallas import tpu as pltpu
from jax.experimental.pallas import tpu_sc as plsc

B, H, W, C = 2, 112, 112, 96
N = B * H * W * C
OUT = B * (2 * H) * (2 * W) * C

NC = 2
NS = 16
NCHUNK = 6
ROUNDS = NCHUNK // NC
CH = OUT // NCHUNK
CPT = CH // NS
SLICE = N // NS
WIN = 7168
NWIN = SLICE // WIN
PAD = 16384


def _body(idx_hbm, val_hbm, out_hbm, idx_win, val_win, acc):
    c = lax.axis_index("c")
    s = lax.axis_index("s")
    zeros = jnp.zeros((16,), jnp.float32)

    for r in range(ROUNDS):
        lo = (c * ROUNDS + r) * CH

        @pl.loop(0, WIN // 16)
        def _(i):
            val_win[pl.ds(i * 16, 16)] = zeros

        for j in range(CPT // WIN):
            pltpu.sync_copy(val_win, acc.at[pl.ds(s * CPT + j * WIN, WIN)])

        plsc.subcore_barrier()

        for w in range(NWIN):
            base = s * SLICE + w * WIN
            pltpu.sync_copy(idx_hbm.at[pl.ds(base, WIN)], idx_win)
            pltpu.sync_copy(val_hbm.at[pl.ds(base, WIN)], val_win)

            @pl.loop(0, WIN // 16)
            def _(i):
                idxv = idx_win[pl.ds(i * 16, 16)]
                rel = idxv - lo
                m = (rel >= 0) & (rel < CH)
                dump = CH + (idxv & (PAD - 1))
                idx_win[pl.ds(i * 16, 16)] = jnp.where(m, rel, dump)

            pltpu.sync_copy(val_win, acc.at[idx_win], add=True)

        plsc.subcore_barrier()
        pltpu.sync_copy(
            acc.at[pl.ds(s * CPT, CPT)],
            out_hbm.at[pl.ds(lo + s * CPT, CPT)],
        )


@jax.jit
def kernel(input, index):
    mesh = plsc.VectorSubcoreMesh(core_axis_name="c", subcore_axis_name="s")
    run = pl.kernel(
        _body,
        out_type=jax.ShapeDtypeStruct((OUT,), jnp.float32),
        mesh=mesh,
        scratch_types=[
            pltpu.VMEM((WIN,), jnp.int32),
            pltpu.VMEM((WIN,), jnp.float32),
            pltpu.VMEM_SHARED((CH + PAD,), jnp.float32),
        ],
    )
    out = run(index.reshape(-1), input.reshape(-1))
    return out.reshape(B, 2 * H, 2 * W, C)
